# Initial kernel scaffold; baseline (speedup 1.0000x reference)
import jax
import jax.numpy as jnp
from jax.experimental import pallas as pl


def kernel(x, topk, W1, b1, W2, b2, W3, b3):
    raise NotImplementedError("write your pallas kernel here")


# speedup vs baseline: 1.5092x; 1.5092x over previous
import functools

import jax
import jax.numpy as jnp
from jax.experimental import pallas as pl

_EPS = 1e-6
_K = 32
_JB = 512


def _sz(z):
    return jnp.where(z >= 0, z + _EPS, z - _EPS)


def _fwd1_body(x_ref, w_ref, b_ref, h_ref, zrow_ref):
    z = jax.lax.dot_general(
        x_ref[...], w_ref[...], (((1,), (1,)), ((), ())),
        preferred_element_type=jnp.float32) + b_ref[...]
    h_ref[...] = jnp.maximum(z, 0.0)
    ones_row = jnp.ones((1, w_ref.shape[1]), jnp.float32)
    zrow_ref[...] = jax.lax.dot_general(
        ones_row, w_ref[...], (((1,), (1,)), ((), ())),
        preferred_element_type=jnp.float32) + b_ref[...]


def _fwd2_body(a_ref, w_ref, b_ref, z_ref, h_ref):
    z = jax.lax.dot_general(
        a_ref[...], w_ref[...], (((1,), (1,)), ((), ())),
        preferred_element_type=jnp.float32) + b_ref[...]
    z_ref[...] = z
    h_ref[...] = jnp.maximum(z, 0.0)


def _fwd3_body(a_ref, w_ref, b_ref, h_ref):
    h_ref[...] = jax.lax.dot_general(
        a_ref[...], w_ref[...], (((1,), (1,)), ((), ())),
        preferred_element_type=jnp.float32) + b_ref[...]


def _topk_body(h3_ref, tk_ref, s3_ref):
    h3 = h3_ref[...]
    m = jnp.max(h3, axis=-1, keepdims=True)
    e = jnp.exp(h3 - m)
    r = e / jnp.sum(e, axis=-1, keepdims=True)
    iota = jax.lax.broadcasted_iota(jnp.int32, h3.shape, 1)
    invz = 1.0 / _sz(h3)
    tk = tk_ref[0, 0]
    work = r
    s3 = jnp.zeros_like(r)
    for k in range(_K):
        cur = jnp.max(work, axis=-1, keepdims=True)
        sel = jnp.min(jnp.where(work == cur, iota, h3.shape[-1]),
                      axis=-1, keepdims=True)
        onehot = iota == sel
        contrib = jnp.where(onehot, r * invz, 0.0)
        s3 = s3 + jnp.where(k < tk, contrib, 0.0)
        work = jnp.where(onehot, -1.0, work)
    s3_ref[...] = s3


def _bwd3_body(s3_ref, w_ref, h2_ref, z2_ref, s2_ref):
    c = jax.lax.dot_general(
        s3_ref[...], w_ref[...], (((1,), (0,)), ((), ())),
        preferred_element_type=jnp.float32)
    s2_ref[...] = h2_ref[...] * c / _sz(z2_ref[...])


def _bwd2_body(s2_ref, w_ref, h1_ref, zrow_ref, s1_ref):
    c = jax.lax.dot_general(
        s2_ref[...], w_ref[...], (((1,), (0,)), ((), ())),
        preferred_element_type=jnp.float32)
    s1_ref[...] = h1_ref[...] * c / _sz(zrow_ref[...])


def _bwd1_body(s1_ref, w_ref, out_ref):
    out_ref[...] = jax.lax.dot_general(
        s1_ref[...], w_ref[...], (((1,), (0,)), ((), ())),
        preferred_element_type=jnp.float32)


def _full(b, d):
    return pl.BlockSpec((b, d), lambda j: (0, 0))


def _colblk(b):
    return pl.BlockSpec((b, _JB), lambda j: (0, j))


def kernel(x, topk, W1, b1, W2, b2, W3, b3):
    B, D = x.shape
    grid = (D // _JB,)
    f32 = jnp.float32
    b1_2d, b2_2d, b3_2d = b1[None, :], b2[None, :], b3[None, :]
    tk = jnp.asarray(topk, jnp.int32).reshape(1, 1)

    wrow_spec = pl.BlockSpec((_JB, D), lambda j: (j, 0))
    wcol_spec = pl.BlockSpec((D, _JB), lambda j: (0, j))

    h1, z1row = pl.pallas_call(
        _fwd1_body, grid=grid,
        in_specs=[_full(B, D), wrow_spec, _colblk(1)],
        out_specs=[_colblk(B), _colblk(1)],
        out_shape=[jax.ShapeDtypeStruct((B, D), f32),
                   jax.ShapeDtypeStruct((1, D), f32)],
    )(x, W1, b1_2d)

    z2, h2 = pl.pallas_call(
        _fwd2_body, grid=grid,
        in_specs=[_full(B, D), wrow_spec, _colblk(1)],
        out_specs=[_colblk(B), _colblk(B)],
        out_shape=[jax.ShapeDtypeStruct((B, D), f32),
                   jax.ShapeDtypeStruct((B, D), f32)],
    )(h1, W2, b2_2d)

    h3 = pl.pallas_call(
        _fwd3_body, grid=grid,
        in_specs=[_full(B, D), wrow_spec, _colblk(1)],
        out_specs=_colblk(B),
        out_shape=jax.ShapeDtypeStruct((B, D), f32),
    )(h2, W3, b3_2d)

    s3 = pl.pallas_call(
        _topk_body,
        out_shape=jax.ShapeDtypeStruct((B, D), f32),
    )(h3, tk)

    s2 = pl.pallas_call(
        _bwd3_body, grid=grid,
        in_specs=[_full(B, D), wcol_spec, _colblk(B), _colblk(B)],
        out_specs=_colblk(B),
        out_shape=jax.ShapeDtypeStruct((B, D), f32),
    )(s3, W3, h2, z2)

    s1 = pl.pallas_call(
        _bwd2_body, grid=grid,
        in_specs=[_full(B, D), wcol_spec, _colblk(B), _colblk(1)],
        out_specs=_colblk(B),
        out_shape=jax.ShapeDtypeStruct((B, D), f32),
    )(s2, W2, h1, z1row)

    out = pl.pallas_call(
        _bwd1_body, grid=grid,
        in_specs=[_full(B, D), wcol_spec],
        out_specs=_colblk(B),
        out_shape=jax.ShapeDtypeStruct((B, D), f32),
    )(s1, W1)

    return out
